# Initial kernel scaffold; baseline (speedup 1.0000x reference)
import jax
import jax.numpy as jnp
from jax.experimental import pallas as pl


def kernel(x, edge_index, orders, W_mp, b_mp, W1, b1, W2, b2, W3, b3, W_out, b_out):
    raise NotImplementedError("write your pallas kernel here")


# speedup vs baseline: 1.6145x; 1.6145x over previous
import functools
import jax
import jax.numpy as jnp
from jax.experimental import pallas as pl
from jax.experimental.pallas import tpu as pltpu

N = 50000
E = 1600000
D = 10
N_ORDERS = 4

NPAD = 51200
BLK = 256
GRID = NPAD // BLK
LW = 16


def _proj_body(x_ref, wu_ref, wv_ref, u_ref, v_ref):
    x = x_ref[...]
    u_ref[...] = jnp.dot(x, wu_ref[...], preferred_element_type=jnp.float32)
    v_ref[...] = jnp.dot(x, wv_ref[...], preferred_element_type=jnp.float32)


def _upd_proj_body(x_ref, u_ref, cnt_ref, s0_ref, s1_ref, wu_ref, wv_ref,
                   xo_ref, uo_ref, vo_ref):
    xn = x_ref[...] + cnt_ref[...] * u_ref[...] + s0_ref[...] + s1_ref[...]
    xo_ref[...] = xn
    uo_ref[...] = jnp.dot(xn, wu_ref[...], preferred_element_type=jnp.float32)
    vo_ref[...] = jnp.dot(xn, wv_ref[...], preferred_element_type=jnp.float32)


def _max_proj_body(u_ref, m0_ref, m1_ref, wu_ref, wv_ref,
                   xo_ref, uo_ref, vo_ref):
    m = jnp.maximum(m0_ref[...], m1_ref[...])
    xn = jnp.where(m > -1e38, u_ref[...] + m, 0.0)
    xo_ref[...] = xn
    uo_ref[...] = jnp.dot(xn, wu_ref[...], preferred_element_type=jnp.float32)
    vo_ref[...] = jnp.dot(xn, wv_ref[...], preferred_element_type=jnp.float32)


def _max_out_body(u_ref, m0_ref, m1_ref, wo_ref, bo_ref, yo_ref):
    m = jnp.maximum(m0_ref[...], m1_ref[...])
    xn = jnp.where(m > -1e38, u_ref[...] + m, 0.0)
    yo_ref[...] = jnp.dot(xn, wo_ref[...],
                          preferred_element_type=jnp.float32) + bo_ref[...]


def _row_spec():
    return pl.BlockSpec((BLK, LW), lambda i: (i, 0))


def _w_spec():
    return pl.BlockSpec((LW, LW), lambda i: (0, 0))


_f32 = jnp.float32


def _proj(x, wu, wv):
    return pl.pallas_call(
        _proj_body,
        grid=(GRID,),
        in_specs=[_row_spec(), _w_spec(), _w_spec()],
        out_specs=[_row_spec(), _row_spec()],
        out_shape=[jax.ShapeDtypeStruct((NPAD, LW), _f32)] * 2,
    )(x, wu, wv)


def _upd_proj(x, u, cnt, s0, s1, wu, wv):
    return pl.pallas_call(
        _upd_proj_body,
        grid=(GRID,),
        in_specs=[_row_spec()] * 5 + [_w_spec(), _w_spec()],
        out_specs=[_row_spec()] * 3,
        out_shape=[jax.ShapeDtypeStruct((NPAD, LW), _f32)] * 3,
    )(x, u, cnt, s0, s1, wu, wv)


def _max_proj(u, m0, m1, wu, wv):
    return pl.pallas_call(
        _max_proj_body,
        grid=(GRID,),
        in_specs=[_row_spec()] * 3 + [_w_spec(), _w_spec()],
        out_specs=[_row_spec()] * 3,
        out_shape=[jax.ShapeDtypeStruct((NPAD, LW), _f32)] * 3,
    )(u, m0, m1, wu, wv)


def _max_out(u, m0, m1, wo, bo):
    return pl.pallas_call(
        _max_out_body,
        grid=(GRID,),
        in_specs=[_row_spec()] * 3 + [_w_spec(),
                                      pl.BlockSpec((1, LW), lambda i: (0, 0))],
        out_specs=_row_spec(),
        out_shape=jax.ShapeDtypeStruct((NPAD, LW), _f32),
    )(u, m0, m1, wo, bo)


def _pad_w(wa, wb, b):
    wu = jnp.zeros((LW, LW), _f32).at[:D, :D].set(wa - wb)
    wv = jnp.zeros((LW, LW), _f32).at[:D, :D].set(wb)
    return wu, wv, jnp.zeros((LW,), _f32).at[:D].set(b)


def _seg_sum(v, src, dst):
    rows = v[src]
    s = jax.ops.segment_sum(rows, dst, num_segments=NPAD)
    return s, jax.ops.segment_sum(jnp.ones_like(dst, _f32), dst,
                                  num_segments=NPAD)


def _seg_max(v, src, dst):
    rows = v[src]
    return jax.ops.segment_max(rows, dst, num_segments=NPAD)


def kernel(x, edge_index, orders, W_mp, b_mp, W1, b1, W2, b2, W3, b3,
           W_out, b_out):
    ei = edge_index.astype(jnp.int32)
    d1, s1 = ei[0], ei[1]
    o = orders.astype(jnp.int32)

    xp = jnp.zeros((NPAD, LW), _f32).at[:N, :D].set(x)

    wu_mp, wv_mp, bp_mp = _pad_w(W_mp[:D], W_mp[D:], b_mp)
    u, v = _proj(xp, wu_mp, wv_mp)
    u = u + bp_mp
    zero16 = jnp.zeros((NPAD, LW), _f32)

    for i in range(N_ORDERS):
        dst = d1[o[i]]
        src = s1[o[i]]
        s, cnt = _seg_sum(v, src, dst)
        if i < N_ORDERS - 1:
            xp, u, v = _upd_proj(xp, u, cnt[:, None] * jnp.ones((1, LW), _f32),
                                 s, zero16, wu_mp, wv_mp)
            u = u + bp_mp
        else:
            wu1, wv1, bp1 = _pad_w(W1[:D], W1[D:], b1)
            xp, u, v = _upd_proj(xp, u, cnt[:, None] * jnp.ones((1, LW), _f32),
                                 s, zero16, wu1, wv1)
            u = u + bp1

    neg = jnp.full((NPAD, LW), -3.0e38, _f32)
    for li, (W, b) in enumerate(((W2, b2), (W3, b3))):
        m = _seg_max(v, d1, s1)
        wua, wva, bpa = _pad_w(W[:D], W[D:], b)
        xp, u, v = _max_proj(u, m, neg, wua, wva)
        u = u + bpa
    m = _seg_max(v, d1, s1)
    wo = jnp.zeros((LW, LW), _f32).at[:D, :D].set(W_out)
    bo = jnp.zeros((1, LW), _f32).at[0, :D].set(b_out)
    y = _max_out(u, m, neg, wo, bo)
    return y[:N, :D]


# speedup vs baseline: 12.6123x; 7.8119x over previous
import functools
import jax
import jax.numpy as jnp
from jax import lax
from jax.experimental import pallas as pl
from jax.experimental.pallas import tpu as pltpu
from jax.experimental.pallas import tpu_sc as plsc

N = 50000
E = 1600000
D = 10
N_ORDERS = 4

NPAD = 51200
BLK = 256
GRID = NPAD // BLK
LW = 16

NC = 2
NS = 16
NW = NC * NS
EC = 2000
RE = E // N_ORDERS


def _proj_body(x_ref, wu_ref, wv_ref, b_ref, u_ref, v_ref):
    x = x_ref[...]
    u_ref[...] = jnp.dot(x, wu_ref[...],
                         preferred_element_type=jnp.float32) + b_ref[...]
    v_ref[...] = jnp.dot(x, wv_ref[...], preferred_element_type=jnp.float32)


def _upd_proj_body(x_ref, u_ref, c0_ref, c1_ref, s0_ref, s1_ref, wu_ref,
                   wv_ref, b_ref, xo_ref, uo_ref, vo_ref):
    cnt = c0_ref[...] + c1_ref[...]
    xn = x_ref[...] + cnt * u_ref[...] + s0_ref[...] + s1_ref[...]
    xo_ref[...] = xn
    uo_ref[...] = jnp.dot(xn, wu_ref[...],
                          preferred_element_type=jnp.float32) + b_ref[...]
    vo_ref[...] = jnp.dot(xn, wv_ref[...], preferred_element_type=jnp.float32)


def _max_proj_body(u_ref, m0_ref, m1_ref, wu_ref, wv_ref, b_ref,
                   xo_ref, uo_ref, vo_ref):
    m = jnp.maximum(m0_ref[...], m1_ref[...])
    xn = jnp.where(m > -1e38, u_ref[...] + m, 0.0)
    xo_ref[...] = xn
    uo_ref[...] = jnp.dot(xn, wu_ref[...],
                          preferred_element_type=jnp.float32) + b_ref[...]
    vo_ref[...] = jnp.dot(xn, wv_ref[...], preferred_element_type=jnp.float32)


def _max_out_body(u_ref, m0_ref, m1_ref, wo_ref, bo_ref, yo_ref):
    m = jnp.maximum(m0_ref[...], m1_ref[...])
    xn = jnp.where(m > -1e38, u_ref[...] + m, 0.0)
    yo_ref[...] = jnp.dot(xn, wo_ref[...],
                          preferred_element_type=jnp.float32) + bo_ref[...]


def _row_spec():
    return pl.BlockSpec((BLK, LW), lambda i: (i, 0))


def _w_spec():
    return pl.BlockSpec((LW, LW), lambda i: (0, 0))


_f32 = jnp.float32


def _b_spec():
    return pl.BlockSpec((1, LW), lambda i: (0, 0))


def _c_spec():
    return pl.BlockSpec((BLK, 1), lambda i: (i, 0))


def _proj(x, wu, wv, b):
    return pl.pallas_call(
        _proj_body,
        grid=(GRID,),
        in_specs=[_row_spec(), _w_spec(), _w_spec(), _b_spec()],
        out_specs=[_row_spec(), _row_spec()],
        out_shape=[jax.ShapeDtypeStruct((NPAD, LW), _f32)] * 2,
    )(x, wu, wv, b)


def _upd_proj(x, u, c0, c1, s0, s1, wu, wv, b):
    return pl.pallas_call(
        _upd_proj_body,
        grid=(GRID,),
        in_specs=[_row_spec(), _row_spec(), _c_spec(), _c_spec(), _row_spec(),
                  _row_spec(), _w_spec(), _w_spec(), _b_spec()],
        out_specs=[_row_spec()] * 3,
        out_shape=[jax.ShapeDtypeStruct((NPAD, LW), _f32)] * 3,
    )(x, u, c0, c1, s0, s1, wu, wv, b)


def _max_proj(u, m0, m1, wu, wv, b):
    return pl.pallas_call(
        _max_proj_body,
        grid=(GRID,),
        in_specs=[_row_spec()] * 3 + [_w_spec(), _w_spec(), _b_spec()],
        out_specs=[_row_spec()] * 3,
        out_shape=[jax.ShapeDtypeStruct((NPAD, LW), _f32)] * 3,
    )(u, m0, m1, wu, wv, b)


def _max_out(u, m0, m1, wo, bo):
    return pl.pallas_call(
        _max_out_body,
        grid=(GRID,),
        in_specs=[_row_spec()] * 3 + [_w_spec(),
                                      pl.BlockSpec((1, LW), lambda i: (0, 0))],
        out_specs=_row_spec(),
        out_shape=jax.ShapeDtypeStruct((NPAD, LW), _f32),
    )(u, m0, m1, wo, bo)


def _pad_w(W, b):
    wa, wb = W[:D], W[D:]
    wu = jnp.zeros((LW, LW), _f32).at[:D, :D].set(wa - wb)
    wv = jnp.zeros((LW, LW), _f32).at[:D, :D].set(wb)
    return wu, wv, jnp.zeros((1, LW), _f32).at[0, :D].set(b)


_mesh = plsc.VectorSubcoreMesh(core_axis_name="c", subcore_axis_name="s")
_sc_params = pltpu.CompilerParams(use_tc_tiling_on_sc=False)


@functools.partial(
    pl.kernel,
    out_type=(jax.ShapeDtypeStruct((E,), jnp.int32),
              jax.ShapeDtypeStruct((E,), jnp.int32),
              jax.ShapeDtypeStruct((NC, N_ORDERS * NPAD), _f32)),
    mesh=_mesh,
    compiler_params=_sc_params,
    scratch_types=[pltpu.VMEM((EC,), jnp.int32),
                   pltpu.VMEM((EC,), jnp.int32),
                   pltpu.VMEM((EC,), jnp.int32),
                   pltpu.VMEM((EC,), jnp.int32),
                   pltpu.VMEM((EC,), _f32),
                   pltpu.VMEM_SHARED((N_ORDERS * NPAD,), _f32)],
)
def _sc_prep(d1_hbm, s1_hbm, ord_hbm, zc_hbm, pdst_hbm, psrc_hbm, cnt_hbm,
             ordi, dsti, srci, idx2, ones, cacc):
    c = lax.axis_index("c")
    s = lax.axis_index("s")
    w = c * NS + s

    sl = N_ORDERS * NPAD // NS
    pltpu.sync_copy(zc_hbm.at[pl.ds(s * sl, sl)], cacc.at[pl.ds(s * sl, sl)])

    @pl.loop(0, EC // 16)
    def _(j):
        ones[pl.ds(j * 16, 16)] = jnp.full((16,), 1.0, _f32)

    plsc.subcore_barrier()

    roff = (w // 8) * NPAD

    @pl.loop(0, 25)
    def _(k):
        off = (w * 25 + k) * EC
        pltpu.sync_copy(ord_hbm.at[pl.ds(off, EC)], ordi)
        pltpu.sync_copy(d1_hbm.at[ordi], dsti)
        pltpu.sync_copy(s1_hbm.at[ordi], srci)
        pltpu.sync_copy(dsti, pdst_hbm.at[pl.ds(off, EC)])
        pltpu.sync_copy(srci, psrc_hbm.at[pl.ds(off, EC)])

        @pl.loop(0, EC // 16)
        def _(j):
            idx2[pl.ds(j * 16, 16)] = dsti[pl.ds(j * 16, 16)] + roff

        pltpu.sync_copy(ones, cacc.at[idx2], add=True)

    plsc.subcore_barrier()
    pltpu.sync_copy(cacc.at[pl.ds(s * sl, sl)],
                    cnt_hbm.at[c].at[pl.ds(s * sl, sl)])


_NCHUNK_R = RE // EC


@functools.partial(
    pl.kernel,
    out_type=jax.ShapeDtypeStruct((NC, NPAD, LW), _f32),
    mesh=_mesh,
    compiler_params=_sc_params,
    scratch_types=[pltpu.VMEM((EC,), jnp.int32),
                   pltpu.VMEM((EC,), jnp.int32),
                   pltpu.VMEM((EC, LW), _f32),
                   pltpu.VMEM_SHARED((NPAD, LW), _f32)],
)
def _sc_round(pdst_hbm, psrc_hbm, v_hbm, zr_hbm, out_hbm, dsti, srci, rows,
              acc):
    c = lax.axis_index("c")
    s = lax.axis_index("s")
    w = c * NS + s

    rsl = NPAD // NS
    pltpu.sync_copy(zr_hbm.at[pl.ds(s * rsl, rsl)], acc.at[pl.ds(s * rsl, rsl)])
    plsc.subcore_barrier()

    for k in range(7):
        m = w + NW * k

        @pl.when(m < _NCHUNK_R)
        def _():
            off = m * EC
            pltpu.sync_copy(pdst_hbm.at[pl.ds(off, EC)], dsti)
            pltpu.sync_copy(psrc_hbm.at[pl.ds(off, EC)], srci)
            pltpu.sync_copy(v_hbm.at[srci], rows)
            pltpu.sync_copy(rows, acc.at[dsti], add=True)

    plsc.subcore_barrier()
    pltpu.sync_copy(acc.at[pl.ds(s * rsl, rsl)],
                    out_hbm.at[c].at[pl.ds(s * rsl, rsl)])


NB = 1600
REG = 50240
WCH = E // NW
CH = 1024
NWR = NW * REG
DUMP = NB
_sc_params_nl = pltpu.CompilerParams(use_tc_tiling_on_sc=False,
                                     needs_layout_passes=False)

_i32 = jnp.int32
_IOTA = lambda: lax.iota(_i32, 16)


def _owner(d16):
    return lax.shift_right_logical(
        lax.shift_right_logical(d16, 6) * 10486, 18)


@functools.partial(
    pl.kernel,
    out_type=(jax.ShapeDtypeStruct((NWR + CH,), _i32),
              jax.ShapeDtypeStruct((NWR + CH,), _i32),
              jax.ShapeDtypeStruct((NW * NW,), _i32),
              jax.ShapeDtypeStruct((NW * NW,), _i32)),
    mesh=_mesh,
    compiler_params=_sc_params_nl,
    scratch_types=[pltpu.VMEM((EC,), _i32),
                   pltpu.VMEM((EC,), _i32),
                   pltpu.VMEM((REG,), _i32),
                   pltpu.VMEM((REG,), _i32),
                   pltpu.VMEM((32,), _i32),
                   pltpu.VMEM((32,), _i32),
                   pltpu.VMEM((32,), _i32)],
)
def _sc_route(dst_hbm, src_hbm, rdst_hbm, rsrc_hbm, st_hbm, en_hbm,
              dsti, srci, gdst, gsrc, hist, base, pidx):
    c = lax.axis_index("c")
    s = lax.axis_index("s")
    w = c * NS + s
    zi = jnp.zeros((16,), _i32)

    hist[pl.ds(0, 16)] = zi
    hist[pl.ds(16, 16)] = zi

    @pl.loop(0, REG // 16)
    def _(j):
        gdst[pl.ds(j * 16, 16)] = zi + DUMP
        gsrc[pl.ds(j * 16, 16)] = _IOTA()

    @pl.when(w == NW - 1)
    def _():
        pltpu.sync_copy(gdst.at[pl.ds(0, CH)], rdst_hbm.at[pl.ds(NWR, CH)])
        pltpu.sync_copy(gsrc.at[pl.ds(0, CH)], rsrc_hbm.at[pl.ds(NWR, CH)])

    @pl.loop(0, WCH // EC)
    def _(k):
        off = w * WCH + k * EC
        pltpu.sync_copy(dst_hbm.at[pl.ds(off, EC)], dsti)

        @pl.loop(0, EC // 16)
        def _(j):
            ow = _owner(dsti[pl.ds(j * 16, 16)])
            plsc.addupdate_scatter(hist, [ow], zi + 1)

    h1 = lax.bitwise_and(hist[pl.ds(0, 16)] + 7, zi + ~7)
    h2 = lax.bitwise_and(hist[pl.ds(16, 16)] + 7, zi + ~7)
    c1 = plsc.cumsum(h1)
    c2 = plsc.cumsum(h2) + jnp.sum(h1)
    base[pl.ds(0, 16)] = c1 - h1
    base[pl.ds(16, 16)] = c2 - h2

    pidx[pl.ds(0, 16)] = _IOTA() * 32 + w
    pidx[pl.ds(16, 16)] = (_IOTA() + 16) * 32 + w
    hist[pl.ds(0, 16)] = c1 - h1 + w * REG
    hist[pl.ds(16, 16)] = c2 - h2 + w * REG
    pltpu.sync_copy(hist, st_hbm.at[pidx])
    hist[pl.ds(0, 16)] = c1 + w * REG
    hist[pl.ds(16, 16)] = c2 + w * REG
    pltpu.sync_copy(hist, en_hbm.at[pidx])

    @pl.loop(0, WCH // EC)
    def _(k):
        off = w * WCH + k * EC
        pltpu.sync_copy(dst_hbm.at[pl.ds(off, EC)], dsti)
        pltpu.sync_copy(src_hbm.at[pl.ds(off, EC)], srci)

        @pl.loop(0, EC // 16)
        def _(j):
            d16 = dsti[pl.ds(j * 16, 16)]
            s16 = srci[pl.ds(j * 16, 16)]
            ow = _owner(d16)
            rank, last = plsc.scan_count(ow)
            b16 = plsc.load_gather(base, [ow])
            pos = b16 + rank - 1
            plsc.store_scatter(base, [ow], pos + 1, mask=last)
            plsc.store_scatter(gdst, [pos], d16 - ow * NB)
            plsc.store_scatter(gsrc, [pos], s16)

    pltpu.sync_copy(gdst, rdst_hbm.at[pl.ds(w * REG, REG)])
    pltpu.sync_copy(gsrc, rsrc_hbm.at[pl.ds(w * REG, REG)])


_NACC = 4
_AROWS = NB + 1


@functools.partial(
    pl.kernel,
    out_type=jax.ShapeDtypeStruct((NPAD * LW,), _f32),
    mesh=_mesh,
    compiler_params=_sc_params_nl,
    scratch_types=[pltpu.VMEM((CH,), _i32),
                   pltpu.VMEM((CH,), _i32),
                   pltpu.VMEM((CH,), _i32),
                   pltpu.VMEM((CH, LW), _f32),
                   pltpu.VMEM((_NACC * _AROWS * LW,), _f32),
                   pltpu.VMEM((32,), _i32),
                   pltpu.VMEM((32,), _i32)],
)
def _sc_maxseg(rdst_hbm, rsrc_hbm, st_hbm, en_hbm, v_hbm, out_hbm,
               dsti, srci, idxb, rows, acc, stv, env):
    c = lax.axis_index("c")
    s = lax.axis_index("s")
    b = c * NS + s
    neg = jnp.full((16,), -3.0e38, _f32)
    zi = jnp.zeros((16,), _i32)
    iota = _IOTA()
    slotpat = lax.bitwise_and(iota, zi + (_NACC - 1)) * _AROWS

    @pl.loop(0, _NACC * _AROWS)
    def _(r):
        acc[pl.ds(r * LW, 16)] = neg

    pltpu.sync_copy(st_hbm.at[pl.ds(b * 32, 32)], stv)
    pltpu.sync_copy(en_hbm.at[pl.ds(b * 32, 32)], env)

    @pl.loop(0, NW)
    def _(wi):
        start = pl.multiple_of(
            jnp.max(plsc.load_gather(stv, [zi + wi])), 8)
        cnt = jnp.max(plsc.load_gather(env, [zi + wi])) - start

        @pl.loop(0, (cnt + CH - 1) // CH)
        def _(k):
            off = pl.multiple_of(start + k * CH, 8)
            rem = jnp.minimum(cnt - k * CH, CH)
            pltpu.sync_copy(rsrc_hbm.at[pl.ds(off, CH)], srci)
            pltpu.sync_copy(v_hbm.at[srci], rows)
            pltpu.sync_copy(rdst_hbm.at[pl.ds(off, CH)], dsti)

            @pl.loop(0, CH // 16)
            def _(q):
                d16 = dsti[pl.ds(q * 16, 16)]
                idxb[pl.ds(q * 16, 16)] = (d16 + slotpat) * LW

            @pl.loop(0, rem // 4)
            def _(g):
                j = g * 4
                i0 = plsc.load_gather(idxb, [zi + j]) + iota
                i1 = plsc.load_gather(idxb, [zi + (j + 1)]) + iota
                i2 = plsc.load_gather(idxb, [zi + (j + 2)]) + iota
                i3 = plsc.load_gather(idxb, [zi + (j + 3)]) + iota
                a0 = plsc.load_gather(acc, [i0])
                a1 = plsc.load_gather(acc, [i1])
                a2 = plsc.load_gather(acc, [i2])
                a3 = plsc.load_gather(acc, [i3])
                plsc.store_scatter(acc, [i0], jnp.maximum(a0, rows[j]))
                plsc.store_scatter(acc, [i1], jnp.maximum(a1, rows[j + 1]))
                plsc.store_scatter(acc, [i2], jnp.maximum(a2, rows[j + 2]))
                plsc.store_scatter(acc, [i3], jnp.maximum(a3, rows[j + 3]))

    @pl.loop(0, NB)
    def _(r):
        m01 = jnp.maximum(acc[pl.ds(r * LW, 16)],
                          acc[pl.ds((_AROWS + r) * LW, 16)])
        m23 = jnp.maximum(acc[pl.ds((2 * _AROWS + r) * LW, 16)],
                          acc[pl.ds((3 * _AROWS + r) * LW, 16)])
        acc[pl.ds(r * LW, 16)] = jnp.maximum(m01, m23)

    pltpu.sync_copy(acc.at[pl.ds(0, NB * LW)],
                    out_hbm.at[pl.ds(b * NB * LW, NB * LW)])


def kernel(x, edge_index, orders, W_mp, b_mp, W1, b1, W2, b2, W3, b3,
           W_out, b_out):
    ei = edge_index.astype(jnp.int32)
    d1 = jnp.reshape(ei[0], (E,))
    s1 = jnp.reshape(ei[1], (E,))
    o = jnp.reshape(orders.astype(jnp.int32), (E,))

    xp = jnp.zeros((NPAD, LW), _f32).at[:N, :D].set(x)
    zc = jnp.zeros((N_ORDERS * NPAD,), _f32)
    zr = jnp.zeros((NPAD, LW), _f32)

    pdst, psrc, cntp = _sc_prep(d1, s1, o, zc)
    cntp = cntp.reshape(NC, N_ORDERS, NPAD, 1)

    rdst, rsrc, est, een = _sc_route(s1, d1)

    wu_mp, wv_mp, bp_mp = _pad_w(W_mp, b_mp)
    u, v = _proj(xp, wu_mp, wv_mp, bp_mp)

    for i in range(N_ORDERS):
        sl = slice(i * RE, (i + 1) * RE)
        S = _sc_round(pdst[sl], psrc[sl], v, zr)
        if i < N_ORDERS - 1:
            wu_n, wv_n, bp_n = wu_mp, wv_mp, bp_mp
        else:
            wu_n, wv_n, bp_n = _pad_w(W1, b1)
        xp, u, v = _upd_proj(xp, u, cntp[0, i], cntp[1, i], S[0], S[1],
                             wu_n, wv_n, bp_n)

    neg = jnp.full((NPAD, LW), -3.0e38, _f32)
    for (W, b) in ((W2, b2), (W3, b3)):
        m = _sc_maxseg(rdst, rsrc, est, een, v).reshape(NPAD, LW)
        wua, wva, bpa = _pad_w(W, b)
        xp, u, v = _max_proj(u, m, neg, wua, wva, bpa)
    m = _sc_maxseg(rdst, rsrc, est, een, v).reshape(NPAD, LW)
    wo = jnp.zeros((LW, LW), _f32).at[:D, :D].set(W_out)
    bo = jnp.zeros((1, LW), _f32).at[0, :D].set(b_out)
    y = _max_out(u, m, neg, wo, bo)
    return y[:N, :D]
